# baseline (speedup 1.0000x reference)
import functools
import math

import jax
import jax.numpy as jnp
from jax import lax
from jax.experimental import pallas as pl
from jax.experimental.pallas import tpu as pltpu
from jax.experimental.pallas import tpu_sc as plsc

NUM_CORES = 2
NUM_SUBCORES = 16
LANES = 16
NW = NUM_CORES * NUM_SUBCORES

_B = 16384
_D = 64
_NI = 10000
_BPW = _B // NW
_CH = 128
_NCH = _BPW // _CH
_NG = _BPW // LANES


def _sigmoid(x):
    return 1.0 / (1.0 + jnp.exp(-x))


def _body(users_ref, pos_ref, neg_ref, ue_ref, ie_ref, gamma_ref, out_ref,
          uidx, pidx, nidx, gpr_idx, gnr_idx, gp_col, gn_col,
          u_rows, p_rows, n_rows, pg_rows, ng_rows, accp, accn, outv, sem):
    cid = lax.axis_index("c")
    sid = lax.axis_index("s")
    wid = sid * NUM_CORES + cid

    pltpu.sync_copy(users_ref.at[wid], uidx)
    pltpu.sync_copy(pos_ref.at[wid], pidx)
    pltpu.sync_copy(neg_ref.at[wid], nidx)

    cps = []
    for k in range(_NCH):
        dst = pl.ds(k * _CH, _CH)
        cps.append(pltpu.async_copy(ue_ref.at[uidx.at[k]], u_rows.at[dst], sem))
        cps.append(pltpu.async_copy(ie_ref.at[pidx.at[k]], p_rows.at[dst], sem))
        cps.append(pltpu.async_copy(ie_ref.at[nidx.at[k]], n_rows.at[dst], sem))

    sh4 = jnp.full((LANES,), 4, jnp.int32)
    m15 = jnp.full((LANES,), 15, jnp.int32)
    for k in range(_NCH):
        for j in range(_CH // LANES):
            sl = pl.ds(j * LANES, LANES)
            uu = uidx[k, sl]
            pp = pidx[k, sl]
            nn = nidx[k, sl]
            fp = uu * _NI + pp
            fn = uu * _NI + nn
            gpr_idx[k, sl] = lax.shift_right_logical(fp, sh4)
            gnr_idx[k, sl] = lax.shift_right_logical(fn, sh4)
            g = k * (_CH // LANES) + j
            gp_col[g, :] = lax.bitwise_and(fp, m15)
            gn_col[g, :] = lax.bitwise_and(fn, m15)
    for k in range(_NCH):
        dst = pl.ds(k * _CH, _CH)
        cps.append(pltpu.async_copy(gamma_ref.at[gpr_idx.at[k]], pg_rows.at[dst], sem))
        cps.append(pltpu.async_copy(gamma_ref.at[gnr_idx.at[k]], ng_rows.at[dst], sem))
    for cp in cps:
        cp.wait()

    iota = lax.iota(jnp.int32, LANES)
    zeros = jnp.zeros((LANES,), jnp.float32)
    c1 = jnp.float32((1e-5 - 1.0) ** 2)
    c0 = jnp.float32(1e-5 ** 2)

    def group(g, carry):
        s1, s2, s3 = carry
        base = g * LANES
        for i in range(LANES):
            srow = base + i
            ap = zeros
            an = zeros
            for jb in range(_D // LANES):
                sl = pl.ds(jb * LANES, LANES)
                uv = u_rows[srow, sl]
                pv = p_rows[srow, sl]
                nv = n_rows[srow, sl]
                ap = ap + uv * pv
                an = an + uv * nv
                s3 = s3 + uv * uv + pv * pv + nv * nv
            accp[i, :] = ap
            accn[i, :] = an
        sp = zeros
        sn = zeros
        for cc in range(LANES):
            col = jnp.full((LANES,), cc, jnp.int32)
            sp = sp + plsc.load_gather(accp, [iota, col])
            sn = sn + plsc.load_gather(accn, [iota, col])
        rowi = base + iota
        gpv = plsc.load_gather(pg_rows, [rowi, gp_col[g, :]])
        gnv = plsc.load_gather(ng_rows, [rowi, gn_col[g, :]])
        sig_p = _sigmoid(sp)
        sig_n = _sigmoid(sn)
        g_p = _sigmoid(gpv)
        g_n = _sigmoid(gnv)
        ep = sig_p - 1.0
        s1 = s1 + g_p * (ep * ep) + g_n * (sig_n * sig_n)
        s2 = s2 + (1.0 - g_p) * c1 + (1.0 - g_n) * c0
        return s1, s2, s3

    s1, s2, s3 = lax.fori_loop(0, _NG, group, (zeros, zeros, zeros))

    w1 = jnp.float32(1.0 / (2.0 * _B))
    w2 = jnp.float32(0.1 / (2.0 * _B))
    w3 = jnp.float32(0.01 * 0.5 / _B)
    outv[:] = s1 * w1 + s2 * w2 + s3 * w3
    pltpu.sync_copy(outv, out_ref.at[wid])


@jax.jit
def _exmf_sc(users_r, pos_r, neg_r, user_embedding, item_embedding, gamma_r):
    kfn = pl.kernel(
        _body,
        out_type=jax.ShapeDtypeStruct((NW, LANES), jnp.float32),
        mesh=plsc.VectorSubcoreMesh(core_axis_name="c", subcore_axis_name="s"),
        compiler_params=pltpu.CompilerParams(
            needs_layout_passes=False, use_tc_tiling_on_sc=False),
        scratch_types=[
            pltpu.VMEM((_NCH, _CH), jnp.int32),
            pltpu.VMEM((_NCH, _CH), jnp.int32),
            pltpu.VMEM((_NCH, _CH), jnp.int32),
            pltpu.VMEM((_NCH, _CH), jnp.int32),
            pltpu.VMEM((_NCH, _CH), jnp.int32),
            pltpu.VMEM((_NG, LANES), jnp.int32),
            pltpu.VMEM((_NG, LANES), jnp.int32),
            pltpu.VMEM((_BPW, _D), jnp.float32),
            pltpu.VMEM((_BPW, _D), jnp.float32),
            pltpu.VMEM((_BPW, _D), jnp.float32),
            pltpu.VMEM((_BPW, LANES), jnp.float32),
            pltpu.VMEM((_BPW, LANES), jnp.float32),
            pltpu.VMEM((LANES, LANES), jnp.float32),
            pltpu.VMEM((LANES, LANES), jnp.float32),
            pltpu.VMEM((LANES,), jnp.float32),
            pltpu.SemaphoreType.DMA,
        ],
    )
    return kfn(users_r, pos_r, neg_r, user_embedding, item_embedding, gamma_r)


def kernel(users, positive_items, negative_items, user_embedding,
           item_embedding, gamma):
    users_r = users.astype(jnp.int32).reshape(NW, _NCH, _CH)
    pos_r = positive_items.astype(jnp.int32).reshape(NW, _NCH, _CH)
    neg_r = negative_items.astype(jnp.int32).reshape(NW, _NCH, _CH)
    gamma_r = gamma.reshape(-1, LANES)
    parts = _exmf_sc(users_r, pos_r, neg_r, user_embedding, item_embedding,
                     gamma_r)
    return jnp.sum(parts)


# speedup vs baseline: 2.7359x; 2.7359x over previous
import jax
import jax.numpy as jnp
from jax import lax
from jax.experimental import pallas as pl
from jax.experimental.pallas import tpu as pltpu
from jax.experimental.pallas import tpu_sc as plsc

NUM_CORES = 2
NUM_SUBCORES = 16
LANES = 16
NW = NUM_CORES * NUM_SUBCORES

_B = 16384
_D = 64
_NI = 10000
_BPW = _B // NW
_CS = 64
_NCHK = _BPW // _CS
_NG = _BPW // LANES
_NSLOT = 3


def _sigmoid(x):
    return 1.0 / (1.0 + jnp.exp(-x))


def _body(users_ref, pos_ref, neg_ref, ue_ref, ie_ref, gamma_ref, out_ref,
          uidx, pidx, nidx, uh, ph, nh, gpv, gnv,
          ubuf, pbuf, nbuf, ring, outv,
          sem_ga, sem_gb, sem_e0, sem_e1, sem_e2):
    cid = lax.axis_index("c")
    sid = lax.axis_index("s")
    wid = sid * NUM_CORES + cid

    iota = lax.iota(jnp.int32, LANES)
    zeros = jnp.zeros((LANES,), jnp.float32)

    pltpu.sync_copy(users_ref.at[wid], uidx)
    pltpu.sync_copy(pos_ref.at[wid], pidx)
    pltpu.sync_copy(neg_ref.at[wid], nidx)
    one = jnp.full((LANES,), 1, jnp.int32)
    for j in range(_BPW // LANES):
        sl = pl.ds(j * LANES, LANES)
        uh[sl] = lax.shift_right_logical(uidx[sl], one)
        ph[sl] = lax.shift_right_logical(pidx[sl], one)
        nh[sl] = lax.shift_right_logical(nidx[sl], one)

    emb_sems = (sem_e0, sem_e1, sem_e2)

    def fire_emb(c, slot, sem):
        src = pl.ds(pl.multiple_of(c * _CS, 8), _CS)
        dst = pl.ds(slot * _CS, _CS)
        pltpu.async_copy(ue_ref.at[uh.at[src]], ubuf.at[dst], sem)
        pltpu.async_copy(ie_ref.at[ph.at[src]], pbuf.at[dst], sem)
        pltpu.async_copy(ie_ref.at[nh.at[src]], nbuf.at[dst], sem)

    def drain_emb(slot, sem):
        dummy = ue_ref.at[pl.ds(0, _CS)]
        dst = pl.ds(slot * _CS, _CS)
        pltpu.make_async_copy(dummy, ubuf.at[dst], sem).wait()
        pltpu.make_async_copy(dummy, pbuf.at[dst], sem).wait()
        pltpu.make_async_copy(dummy, nbuf.at[dst], sem).wait()

    for c in range(_NSLOT):
        fire_emb(c, c, emb_sems[c])

    m7 = jnp.full((LANES,), 7, jnp.int32)
    m127 = jnp.full((LANES,), 127, jnp.int32)

    def issue_gamma(items_ref, g, halfbase, sem):
        uvec = uidx[pl.ds(g * LANES, LANES)]
        ivec = items_ref[pl.ds(g * LANES, LANES)]
        for l in range(LANES):
            u = uvec[l]
            it = ivec[l]
            r8 = pl.multiple_of((u >> 3) << 3, 8)
            c128 = pl.multiple_of((it >> 7) << 7, 128)
            pltpu.async_copy(
                gamma_ref.at[pl.ds(r8, 8), pl.ds(c128, 128)],
                ring.at[pl.ds((halfbase + l) * 8, 8)], sem)

    def drain_gamma(halfbase, sem):
        dummy = gamma_ref.at[pl.ds(0, 128), pl.ds(0, 128)]
        pltpu.make_async_copy(
            dummy, ring.at[pl.ds(halfbase * 8, 128)], sem).wait()

    def extract_gamma(items_ref, g, halfbase, dst_ref):
        uvec = uidx[pl.ds(g * LANES, LANES)]
        ivec = items_ref[pl.ds(g * LANES, LANES)]
        rvec = (iota + halfbase) * 8 + lax.bitwise_and(uvec, m7)
        cvec = lax.bitwise_and(ivec, m127)
        dst_ref[pl.ds(g * LANES, LANES)] = plsc.load_gather(ring, [rvec, cvec])

    issue_gamma(pidx, 0, 0, sem_ga)
    issue_gamma(nidx, 0, LANES, sem_gb)

    def gamma_step(g, carry):
        drain_gamma(0, sem_ga)
        extract_gamma(pidx, g, 0, gpv)
        issue_gamma(pidx, g + 1, 0, sem_ga)
        drain_gamma(LANES, sem_gb)
        extract_gamma(nidx, g, LANES, gnv)
        issue_gamma(nidx, g + 1, LANES, sem_gb)
        return carry

    lax.fori_loop(0, _NG - 1, gamma_step, 0)
    drain_gamma(0, sem_ga)
    extract_gamma(pidx, _NG - 1, 0, gpv)
    drain_gamma(LANES, sem_gb)
    extract_gamma(nidx, _NG - 1, LANES, gnv)

    c1 = jnp.float32((1e-5 - 1.0) ** 2)
    c0 = jnp.float32(1e-5 ** 2)
    m1 = jnp.full((LANES,), 1, jnp.int32)

    def group_step(g, carry):
        s1, s2, sq = carry
        c = g >> 2
        slot = lax.rem(c, _NSLOT)

        @pl.when(lax.rem(g, jnp.int32(4)) == 0)
        def _chunk_boundary():
            for s in range(_NSLOT):
                @pl.when(lax.rem(c, _NSLOT) == s)
                def _per_slot():
                    drain_emb(s, emb_sems[s])

                    @pl.when(c + _NSLOT < _NCHK)
                    def _refire():
                        fire_emb(c + _NSLOT, s, emb_sems[s])

        base = g * LANES
        uvec = uidx[pl.ds(base, LANES)]
        pvec = pidx[pl.ds(base, LANES)]
        nvec = nidx[pl.ds(base, LANES)]
        paru = lax.shift_left(lax.bitwise_and(uvec, m1), jnp.full((LANES,), 6, jnp.int32))
        parp = lax.shift_left(lax.bitwise_and(pvec, m1), jnp.full((LANES,), 6, jnp.int32))
        parn = lax.shift_left(lax.bitwise_and(nvec, m1), jnp.full((LANES,), 6, jnp.int32))
        rows = iota + slot * _CS + lax.rem(g, jnp.int32(4)) * LANES

        accp = zeros
        accn = zeros
        for d in range(_D):
            gu = plsc.load_gather(ubuf, [rows, paru + d])
            gp = plsc.load_gather(pbuf, [rows, parp + d])
            gn = plsc.load_gather(nbuf, [rows, parn + d])
            accp = accp + gu * gp
            accn = accn + gu * gn
            sq = sq + gu * gu + gp * gp + gn * gn

        sig_p = _sigmoid(accp)
        sig_n = _sigmoid(accn)
        g_p = _sigmoid(gpv[pl.ds(base, LANES)])
        g_n = _sigmoid(gnv[pl.ds(base, LANES)])
        ep = sig_p - 1.0
        s1 = s1 + g_p * (ep * ep) + g_n * (sig_n * sig_n)
        s2 = s2 + (1.0 - g_p) * c1 + (1.0 - g_n) * c0
        return s1, s2, sq

    s1, s2, sq = lax.fori_loop(0, _NG, group_step, (zeros, zeros, zeros))

    w1 = jnp.float32(1.0 / (2.0 * _B))
    w2 = jnp.float32(0.1 / (2.0 * _B))
    w3 = jnp.float32(0.01 * 0.5 / _B)
    outv[:] = s1 * w1 + s2 * w2 + sq * w3
    pltpu.sync_copy(outv, out_ref.at[wid])


@jax.jit
def _exmf_sc(users_r, pos_r, neg_r, ue2, ie2, gamma):
    kfn = pl.kernel(
        _body,
        out_type=jax.ShapeDtypeStruct((NW, LANES), jnp.float32),
        mesh=plsc.VectorSubcoreMesh(core_axis_name="c", subcore_axis_name="s"),
        compiler_params=pltpu.CompilerParams(needs_layout_passes=False),
        scratch_types=[
            pltpu.VMEM((_BPW,), jnp.int32),
            pltpu.VMEM((_BPW,), jnp.int32),
            pltpu.VMEM((_BPW,), jnp.int32),
            pltpu.VMEM((_BPW,), jnp.int32),
            pltpu.VMEM((_BPW,), jnp.int32),
            pltpu.VMEM((_BPW,), jnp.int32),
            pltpu.VMEM((_BPW,), jnp.float32),
            pltpu.VMEM((_BPW,), jnp.float32),
            pltpu.VMEM((_NSLOT * _CS, 128), jnp.float32),
            pltpu.VMEM((_NSLOT * _CS, 128), jnp.float32),
            pltpu.VMEM((_NSLOT * _CS, 128), jnp.float32),
            pltpu.VMEM((2 * LANES * 8, 128), jnp.float32),
            pltpu.VMEM((LANES,), jnp.float32),
            pltpu.SemaphoreType.DMA,
            pltpu.SemaphoreType.DMA,
            pltpu.SemaphoreType.DMA,
            pltpu.SemaphoreType.DMA,
            pltpu.SemaphoreType.DMA,
        ],
    )
    return kfn(users_r, pos_r, neg_r, ue2, ie2, gamma)


def kernel(users, positive_items, negative_items, user_embedding,
           item_embedding, gamma):
    users_r = users.astype(jnp.int32).reshape(NW, _BPW)
    pos_r = positive_items.astype(jnp.int32).reshape(NW, _BPW)
    neg_r = negative_items.astype(jnp.int32).reshape(NW, _BPW)
    ue2 = user_embedding.reshape(_NI // 2, 2 * _D)
    ie2 = item_embedding.reshape(_NI // 2, 2 * _D)
    parts = _exmf_sc(users_r, pos_r, neg_r, ue2, ie2, gamma)
    return jnp.sum(parts)


# speedup vs baseline: 3.9677x; 1.4503x over previous
import jax
import jax.numpy as jnp
from jax import lax
from jax.experimental import pallas as pl
from jax.experimental.pallas import tpu as pltpu
from jax.experimental.pallas import tpu_sc as plsc

NUM_CORES = 2
NUM_SUBCORES = 16
LANES = 16
NW = NUM_CORES * NUM_SUBCORES

_B = 16384
_D = 64
_NI = 10000
_BPW = _B // NW
_CS = 64
_NCHK = _BPW // _CS
_NG = _BPW // LANES
_HALF = 2 * LANES * 8


def _sigmoid(x):
    return 1.0 / (1.0 + jnp.exp(-x))


def _tree_sum(vals):
    while len(vals) > 1:
        vals = [a + b for a, b in zip(vals[::2], vals[1::2])]
    return vals[0]


def _body(users_ref, pos_ref, neg_ref, ue_ref, ie_ref, gamma_ref, out_ref,
          uidx, pidx, nidx, ubuf, pbuf, nbuf, ring, accp, accn, outv,
          sem_ga, sem_gb, sem_e0, sem_e1):
    cid = lax.axis_index("c")
    sid = lax.axis_index("s")
    wid = sid * NUM_CORES + cid

    iota = lax.iota(jnp.int32, LANES)
    zeros = jnp.zeros((LANES,), jnp.float32)
    m7 = jnp.full((LANES,), 7, jnp.int32)
    m127 = jnp.full((LANES,), 127, jnp.int32)
    s3 = jnp.full((LANES,), 3, jnp.int32)
    s7 = jnp.full((LANES,), 7, jnp.int32)

    pltpu.sync_copy(users_ref.at[wid], uidx)
    pltpu.sync_copy(pos_ref.at[wid], pidx)
    pltpu.sync_copy(neg_ref.at[wid], nidx)

    def fire_emb(c, slot, sem):
        src = pl.ds(pl.multiple_of(c * _CS, 8), _CS)
        dst = pl.ds(slot * _CS, _CS)
        pltpu.async_copy(ue_ref.at[uidx.at[src]], ubuf.at[dst], sem)
        pltpu.async_copy(ie_ref.at[pidx.at[src]], pbuf.at[dst], sem)
        pltpu.async_copy(ie_ref.at[nidx.at[src]], nbuf.at[dst], sem)

    def drain_emb(slot, sem):
        dummy = ue_ref.at[pl.ds(0, _CS)]
        dst = pl.ds(slot * _CS, _CS)
        pltpu.make_async_copy(dummy, ubuf.at[dst], sem).wait()
        pltpu.make_async_copy(dummy, pbuf.at[dst], sem).wait()
        pltpu.make_async_copy(dummy, nbuf.at[dst], sem).wait()

    def issue_gamma(g, halfrow, sem):
        for items_ref, sub in ((pidx, 0), (nidx, LANES)):
            uvec = uidx[pl.ds(g * LANES, LANES)]
            ivec = items_ref[pl.ds(g * LANES, LANES)]
            r8v = lax.shift_left(lax.shift_right_logical(uvec, s3), s3)
            c128v = lax.shift_left(lax.shift_right_logical(ivec, s7), s7)
            for l in range(LANES):
                r8 = pl.multiple_of(r8v[l], 8)
                c128 = pl.multiple_of(c128v[l], 128)
                pltpu.async_copy(
                    gamma_ref.at[pl.ds(r8, 8), pl.ds(c128, 128)],
                    ring.at[pl.ds(halfrow + (sub + l) * 8, 8)], sem)

    def drain_gamma(halfrow, sem):
        dummy = gamma_ref.at[pl.ds(0, _HALF), pl.ds(0, 128)]
        pltpu.make_async_copy(
            dummy, ring.at[pl.ds(halfrow, _HALF)], sem).wait()

    def extract_gamma(items_ref, g, halfrow, sub):
        uvec = uidx[pl.ds(g * LANES, LANES)]
        ivec = items_ref[pl.ds(g * LANES, LANES)]
        rvec = (iota + sub) * 8 + halfrow + lax.bitwise_and(uvec, m7)
        cvec = lax.bitwise_and(ivec, m127)
        return plsc.load_gather(ring, [rvec, cvec])

    fire_emb(0, 0, sem_e0)
    fire_emb(1, 1, sem_e1)
    issue_gamma(0, 0, sem_ga)
    issue_gamma(1, _HALF, sem_gb)

    c1 = jnp.float32((1e-5 - 1.0) ** 2)
    c0 = jnp.float32(1e-5 ** 2)

    def group_step(g, carry):
        s1, s2, sq = carry
        c = g >> 2
        gm8 = lax.rem(g, jnp.int32(8))

        @pl.when(gm8 == 0)
        def _():
            drain_emb(0, sem_e0)

        @pl.when(gm8 == 4)
        def _():
            drain_emb(1, sem_e1)

        @pl.when(jnp.logical_and(gm8 == 3, c + 2 < _NCHK))
        def _():
            fire_emb(c + 2, 0, sem_e0)

        @pl.when(jnp.logical_and(gm8 == 7, c + 2 < _NCHK))
        def _():
            fire_emb(c + 2, 1, sem_e1)

        godd = lax.rem(g, jnp.int32(2))

        @pl.when(godd == 0)
        def _():
            drain_gamma(0, sem_ga)

        @pl.when(godd == 1)
        def _():
            drain_gamma(_HALF, sem_gb)

        halfrow = godd * _HALF
        gpraw = extract_gamma(pidx, g, halfrow, 0)
        gnraw = extract_gamma(nidx, g, halfrow, LANES)

        @pl.when(jnp.logical_and(godd == 0, g + 2 < _NG))
        def _():
            issue_gamma(g + 2, 0, sem_ga)

        @pl.when(jnp.logical_and(godd == 1, g + 2 < _NG))
        def _():
            issue_gamma(g + 2, _HALF, sem_gb)

        slot = lax.bitwise_and(c, jnp.int32(1))
        rowbase = slot * _CS + lax.rem(g, jnp.int32(4)) * LANES
        for l in range(LANES):
            row = rowbase + l
            ap = None
            an = None
            for j in range(_D // LANES):
                sl = pl.ds(j * LANES, LANES)
                uv = ubuf[row, sl]
                pv = pbuf[row, sl]
                nv = nbuf[row, sl]
                if ap is None:
                    ap = uv * pv
                    an = uv * nv
                else:
                    ap = ap + uv * pv
                    an = an + uv * nv
                sq = sq + uv * uv + pv * pv + nv * nv
            lcol = jnp.full((LANES,), l, jnp.int32)
            plsc.store_scatter(accp, [iota, lcol], ap)
            plsc.store_scatter(accn, [iota, lcol], an)
        sp = _tree_sum([accp[r] for r in range(LANES)])
        sn = _tree_sum([accn[r] for r in range(LANES)])

        sig_p = _sigmoid(sp)
        sig_n = _sigmoid(sn)
        g_p = _sigmoid(gpraw)
        g_n = _sigmoid(gnraw)
        ep = sig_p - 1.0
        s1 = s1 + g_p * (ep * ep) + g_n * (sig_n * sig_n)
        s2 = s2 + (1.0 - g_p) * c1 + (1.0 - g_n) * c0
        return s1, s2, sq

    s1, s2, sq = lax.fori_loop(0, _NG, group_step, (zeros, zeros, zeros))

    w1 = jnp.float32(1.0 / (2.0 * _B))
    w2 = jnp.float32(0.1 / (2.0 * _B))
    w3 = jnp.float32(0.01 * 0.5 / _B)
    outv[:] = s1 * w1 + s2 * w2 + sq * w3
    pltpu.sync_copy(outv, out_ref.at[wid])


@jax.jit
def _exmf_sc(users_r, pos_r, neg_r, ue_p, ie_p, gamma):
    kfn = pl.kernel(
        _body,
        out_type=jax.ShapeDtypeStruct((NW, LANES), jnp.float32),
        mesh=plsc.VectorSubcoreMesh(core_axis_name="c", subcore_axis_name="s"),
        compiler_params=pltpu.CompilerParams(needs_layout_passes=False),
        scratch_types=[
            pltpu.VMEM((_BPW,), jnp.int32),
            pltpu.VMEM((_BPW,), jnp.int32),
            pltpu.VMEM((_BPW,), jnp.int32),
            pltpu.VMEM((2 * _CS, 128), jnp.float32),
            pltpu.VMEM((2 * _CS, 128), jnp.float32),
            pltpu.VMEM((2 * _CS, 128), jnp.float32),
            pltpu.VMEM((2 * _HALF, 128), jnp.float32),
            pltpu.VMEM((LANES, LANES), jnp.float32),
            pltpu.VMEM((LANES, LANES), jnp.float32),
            pltpu.VMEM((LANES,), jnp.float32),
            pltpu.SemaphoreType.DMA,
            pltpu.SemaphoreType.DMA,
            pltpu.SemaphoreType.DMA,
            pltpu.SemaphoreType.DMA,
        ],
    )
    return kfn(users_r, pos_r, neg_r, ue_p, ie_p, gamma)


def kernel(users, positive_items, negative_items, user_embedding,
           item_embedding, gamma):
    users_r = users.astype(jnp.int32).reshape(NW, _BPW)
    pos_r = positive_items.astype(jnp.int32).reshape(NW, _BPW)
    neg_r = negative_items.astype(jnp.int32).reshape(NW, _BPW)
    ue_p = jnp.pad(user_embedding, ((0, 0), (0, 128 - _D)))
    ie_p = jnp.pad(item_embedding, ((0, 0), (0, 128 - _D)))
    parts = _exmf_sc(users_r, pos_r, neg_r, ue_p, ie_p, gamma)
    return jnp.sum(parts)
